# baseline (speedup 1.0000x reference)
import functools

import jax
import jax.numpy as jnp
from jax import lax
from jax.experimental import pallas as pl
from jax.experimental.pallas import tpu as pltpu
from jax.experimental.pallas import tpu_sc as plsc

N_NODES = 100000
N_EDGES = 3200000
NUM_GRAPHS = 64
HIDDEN = 32
NUM_CLASSES = 2

NC = 2
NS = 16
CHUNK = 128
IDX_ROWS = 16
EBLK = IDX_ROWS * CHUNK

IDX_ROWS2 = 10

_E_ALIGN = 327680
E_PAD = ((N_EDGES + _E_ALIGN - 1) // _E_ALIGN) * _E_ALIGN
E_ROWS = E_PAD // CHUNK
DUMP = N_NODES
N_TAB = 100352
ROWS_PER_SUB = N_TAB // NS
R_TC = 2048
NB_TC = N_TAB // R_TC

_mesh = functools.partial(
    plsc.VectorSubcoreMesh, core_axis_name="c", subcore_axis_name="s",
    num_cores=NC, num_subcores=NS)
_SC_PARAMS = pltpu.CompilerParams(use_tc_tiling_on_sc=False)

def _pipelined_edge_block(tab_hbm, idx_s, idx_d, rows_v, acc_sh, gsem, ssem,
                          k, nrows):
    ngroup = nrows // k
    sgroups = [None] * ngroup
    gh = [pltpu.async_copy(tab_hbm.at[idx_s.at[t]], rows_v.at[t], gsem)
          for t in range(k)]
    for g in range(ngroup):
        base, boff = g * k, (g % 2) * k
        for h in gh:
            h.wait()
        sgroups[g] = [
            pltpu.async_copy(rows_v.at[boff + t],
                             acc_sh.at[idx_d.at[base + t]], ssem, add=True)
            for t in range(k)]
        if g + 1 < ngroup:
            nboff = ((g + 1) % 2) * k
            if g >= 1:
                for h in sgroups[g - 1]:
                    h.wait()
            gh = [pltpu.async_copy(tab_hbm.at[idx_s.at[base + k + t]],
                                   rows_v.at[nboff + t], gsem)
                  for t in range(k)]
    for g in range(max(0, ngroup - 2), ngroup):
        for h in sgroups[g]:
            h.wait()


@functools.lru_cache(maxsize=1)
def _sc_kernels():
    @functools.partial(
        pl.kernel,
        out_type=jax.ShapeDtypeStruct((NC, N_TAB), jnp.float32),
        mesh=_mesh(),
        compiler_params=_SC_PARAMS,
        scratch_types=[
            pltpu.VMEM((IDX_ROWS, CHUNK), jnp.int32),
            pltpu.VMEM((CHUNK,), jnp.float32),
            pltpu.VMEM_SHARED((N_TAB,), jnp.float32),
            pltpu.SemaphoreType.DMA,
        ],
    )
    def _sc_degree(src_hbm, zeros1_hbm, out_hbm, idx_v, ones_v, deg_sh, ssem):
        c = lax.axis_index("c")
        s = lax.axis_index("s")
        wid = s * NC + c

        for k in range(CHUNK // 16):
            ones_v[pl.ds(k * 16, 16)] = jnp.ones((16,), jnp.float32)

        pltpu.sync_copy(zeros1_hbm, deg_sh.at[pl.ds(s * ROWS_PER_SUB, ROWS_PER_SUB)])
        plsc.subcore_barrier()

        blocks = E_ROWS // (NC * NS)
        nblk = blocks // IDX_ROWS

        def body(b, _):
            row0 = wid * blocks + b * IDX_ROWS
            pltpu.sync_copy(src_hbm.at[pl.ds(row0, IDX_ROWS)], idx_v)
            hs = [pltpu.async_copy(ones_v, deg_sh.at[idx_v.at[j]], ssem,
                                   add=True) for j in range(IDX_ROWS)]
            for h in hs:
                h.wait()
            return 0

        lax.fori_loop(0, nblk, body, 0)
        plsc.subcore_barrier()
        pltpu.sync_copy(deg_sh.at[pl.ds(s * ROWS_PER_SUB, ROWS_PER_SUB)],
                        out_hbm.at[c, pl.ds(s * ROWS_PER_SUB, ROWS_PER_SUB)])


    @functools.partial(
        pl.kernel,
        out_type=jax.ShapeDtypeStruct((NC, N_TAB, 8), jnp.float32),
        mesh=_mesh(),
        compiler_params=_SC_PARAMS,
        scratch_types=[
            pltpu.VMEM((IDX_ROWS, CHUNK), jnp.int32),
            pltpu.VMEM((IDX_ROWS, CHUNK), jnp.int32),
            pltpu.VMEM((IDX_ROWS, CHUNK, 8), jnp.float32),
            pltpu.VMEM_SHARED((N_TAB, 8), jnp.float32),
            pltpu.SemaphoreType.DMA,
            pltpu.SemaphoreType.DMA,
        ],
    )
    def _sc_edge1(src_hbm, dst_hbm, g1_hbm, zeros8_hbm, out_hbm,
                  idx_s, idx_d, rows_v, acc_sh, gsem, ssem):
        c = lax.axis_index("c")
        s = lax.axis_index("s")
        wid = s * NC + c

        pltpu.sync_copy(zeros8_hbm, acc_sh.at[pl.ds(s * ROWS_PER_SUB, ROWS_PER_SUB)])
        plsc.subcore_barrier()

        blocks = E_ROWS // (NC * NS)
        nblk = blocks // IDX_ROWS

        def body(b, _):
            row0 = wid * blocks + b * IDX_ROWS
            pltpu.sync_copy(src_hbm.at[pl.ds(row0, IDX_ROWS)], idx_s)
            pltpu.sync_copy(dst_hbm.at[pl.ds(row0, IDX_ROWS)], idx_d)
            _pipelined_edge_block(g1_hbm, idx_s, idx_d, rows_v, acc_sh,
                                  gsem, ssem, 8, IDX_ROWS)
            return 0

        lax.fori_loop(0, nblk, body, 0)
        plsc.subcore_barrier()
        pltpu.sync_copy(acc_sh.at[pl.ds(s * ROWS_PER_SUB, ROWS_PER_SUB)],
                        out_hbm.at[c, pl.ds(s * ROWS_PER_SUB, ROWS_PER_SUB)])


    @functools.partial(
        pl.kernel,
        out_type=jax.ShapeDtypeStruct((NC, N_TAB, 16), jnp.float32),
        mesh=_mesh(),
        compiler_params=_SC_PARAMS,
        scratch_types=[
            pltpu.VMEM((IDX_ROWS2, CHUNK), jnp.int32),
            pltpu.VMEM((IDX_ROWS2, CHUNK), jnp.int32),
            pltpu.VMEM((IDX_ROWS2, CHUNK, 16), jnp.float32),
            pltpu.VMEM_SHARED((N_TAB, 16), jnp.float32),
            pltpu.SemaphoreType.DMA,
            pltpu.SemaphoreType.DMA,
        ],
    )
    def _sc_edge2(src_hbm, dst_hbm, g2a_hbm, g2b_hbm, zeros16_hbm, out_hbm,
                  idx_s, idx_d, rows_v, acc_sh, gsem, ssem):
        c = lax.axis_index("c")
        s = lax.axis_index("s")

        pltpu.sync_copy(zeros16_hbm, acc_sh.at[pl.ds(s * ROWS_PER_SUB, ROWS_PER_SUB)])
        plsc.subcore_barrier()

        blocks = E_ROWS // NS
        nblk = blocks // IDX_ROWS2

        def run(tab_hbm):
            def body(b, _):
                row0 = s * blocks + b * IDX_ROWS2
                pltpu.sync_copy(src_hbm.at[pl.ds(row0, IDX_ROWS2)], idx_s)
                pltpu.sync_copy(dst_hbm.at[pl.ds(row0, IDX_ROWS2)], idx_d)
                _pipelined_edge_block(tab_hbm, idx_s, idx_d, rows_v, acc_sh,
                                      gsem, ssem, 5, IDX_ROWS2)
                return 0
            lax.fori_loop(0, nblk, body, 0)

        @pl.when(c == 0)
        def _():
            run(g2a_hbm)

        @pl.when(c == 1)
        def _():
            run(g2b_hbm)

        plsc.subcore_barrier()
        pltpu.sync_copy(acc_sh.at[pl.ds(s * ROWS_PER_SUB, ROWS_PER_SUB)],
                        out_hbm.at[c, pl.ds(s * ROWS_PER_SUB, ROWS_PER_SUB)])


    return _sc_degree, _sc_edge1, _sc_edge2


def _tc_prep_body(degt_ref, pos_ref, g1_ref):
    deg = degt_ref[:, 0:1] + degt_ref[:, 1:2] + 1.0
    dinv = lax.rsqrt(deg)
    g1_ref[...] = jnp.concatenate(
        [pos_ref[...] * dinv, dinv, jnp.zeros_like(pos_ref[...]), dinv], axis=1)


def _tc_prep(degt, posp):
    return pl.pallas_call(
        _tc_prep_body,
        grid=(NB_TC,),
        in_specs=[
            pl.BlockSpec((R_TC, 2), lambda i: (i, 0)),
            pl.BlockSpec((R_TC, 3), lambda i: (i, 0)),
        ],
        out_specs=pl.BlockSpec((R_TC, 8), lambda i: (i, 0)),
        out_shape=jax.ShapeDtypeStruct((N_TAB, 8), jnp.float32),
    )(degt, posp)


def _tc_node1_body(s1p_ref, g1_ref, w1_ref, b1_ref, g2a_ref, g2b_ref):
    g1 = g1_ref[...]
    dinv = g1[:, 3:4]
    aggr = dinv * (s1p_ref[0] + s1p_ref[1] + g1)
    h1 = lax.dot_general(aggr, w1_ref[...], (((1,), (1,)), ((), ())),
                         preferred_element_type=jnp.float32,
                         precision=lax.Precision.HIGHEST)
    h1 = jnp.maximum(h1 + b1_ref[...], 0.0)
    g2 = h1 * dinv
    g2a_ref[...] = g2[:, :16]
    g2b_ref[...] = g2[:, 16:]


def _tc_node1(s1p, g1, w1p, b1r):
    return pl.pallas_call(
        _tc_node1_body,
        grid=(NB_TC,),
        in_specs=[
            pl.BlockSpec((NC, R_TC, 8), lambda i: (0, i, 0)),
            pl.BlockSpec((R_TC, 8), lambda i: (i, 0)),
            pl.BlockSpec((HIDDEN, 8), lambda i: (0, 0)),
            pl.BlockSpec((1, HIDDEN), lambda i: (0, 0)),
        ],
        out_specs=[
            pl.BlockSpec((R_TC, 16), lambda i: (i, 0)),
            pl.BlockSpec((R_TC, 16), lambda i: (i, 0)),
        ],
        out_shape=[
            jax.ShapeDtypeStruct((N_TAB, 16), jnp.float32),
            jax.ShapeDtypeStruct((N_TAB, 16), jnp.float32),
        ],
    )(s1p, g1, w1p, b1r)


def _tc_node2_body(lohi_ref, s2p_ref, g2a_ref, g2b_ref, g1_ref, batch_ref,
                   w2_ref, b2_ref, wc_ref, bc_ref, out_ref, pool_ref):
    i = pl.program_id(0)

    @pl.when(i == 0)
    def _():
        pool_ref[...] = jnp.full((NUM_GRAPHS, 2 * HIDDEN), -jnp.inf,
                                 jnp.float32)

    g2 = jnp.concatenate([g2a_ref[...], g2b_ref[...]], axis=1)
    s2 = jnp.concatenate([s2p_ref[0], s2p_ref[1]], axis=1)
    dinv = g1_ref[:, 3:4]
    aggr = dinv * (s2 + g2)
    h2 = lax.dot_general(aggr, w2_ref[...], (((1,), (1,)), ((), ())),
                         preferred_element_type=jnp.float32,
                         precision=lax.Precision.HIGHEST)
    h2 = jnp.maximum(h2 + b2_ref[...], 0.0)

    bb = batch_ref[...]
    lo = lohi_ref[i, 0]
    hi = lohi_ref[i, 1]
    for g in range(NUM_GRAPHS):
        @pl.when((lo <= g) & (g <= hi))
        def _():
            sel = bb == g
            m = jnp.max(jnp.where(sel, h2, -jnp.inf), axis=0, keepdims=True)
            pool_ref[g:g + 1, :] = jnp.maximum(pool_ref[g:g + 1, :], m)

    @pl.when(i == NB_TC - 1)
    def _():
        out = lax.dot_general(pool_ref[...], wc_ref[...],
                              (((1,), (1,)), ((), ())),
                              preferred_element_type=jnp.float32,
                              precision=lax.Precision.HIGHEST)
        out_ref[...] = out + bc_ref[...]


def _tc_node2(lohi, s2p, g2a, g2b, g1, batchp, w2, b2r, wc, bcr):
    return pl.pallas_call(
        _tc_node2_body,
        grid=(NB_TC,),
        in_specs=[
            pl.BlockSpec(memory_space=pltpu.SMEM),
            pl.BlockSpec((NC, R_TC, 16), lambda i: (0, i, 0)),
            pl.BlockSpec((R_TC, 16), lambda i: (i, 0)),
            pl.BlockSpec((R_TC, 16), lambda i: (i, 0)),
            pl.BlockSpec((R_TC, 8), lambda i: (i, 0)),
            pl.BlockSpec((R_TC, 1), lambda i: (i, 0)),
            pl.BlockSpec((2 * HIDDEN, HIDDEN), lambda i: (0, 0)),
            pl.BlockSpec((1, 2 * HIDDEN), lambda i: (0, 0)),
            pl.BlockSpec((NUM_CLASSES, 2 * HIDDEN), lambda i: (0, 0)),
            pl.BlockSpec((1, NUM_CLASSES), lambda i: (0, 0)),
        ],
        out_specs=pl.BlockSpec((NUM_GRAPHS, NUM_CLASSES), lambda i: (0, 0)),
        out_shape=jax.ShapeDtypeStruct((NUM_GRAPHS, NUM_CLASSES), jnp.float32),
        scratch_shapes=[pltpu.VMEM((NUM_GRAPHS, 2 * HIDDEN), jnp.float32)],
    )(lohi, s2p, g2a, g2b, g1, batchp, w2, b2r, wc, bcr)


def kernel(pos, edge_index, batch, W1, b1, W2, b2, Wc, bc):
    pad_e = E_PAD - N_EDGES
    src = jnp.concatenate(
        [edge_index[0].astype(jnp.int32),
         jnp.full((pad_e,), DUMP, jnp.int32)]).reshape(E_ROWS, CHUNK)
    dst = jnp.concatenate(
        [edge_index[1].astype(jnp.int32),
         jnp.full((pad_e,), DUMP, jnp.int32)]).reshape(E_ROWS, CHUNK)
    posp = jnp.pad(pos, ((0, N_TAB - N_NODES), (0, 0)))
    batchp = jnp.pad(batch.astype(jnp.int32), (0, N_TAB - N_NODES),
                     constant_values=NUM_GRAPHS).reshape(N_TAB, 1)
    lohi = jnp.stack([batchp[::R_TC, 0], batchp[R_TC - 1::R_TC, 0]], axis=1)
    w1p = jnp.pad(W1, ((0, 0), (0, 5)))
    b1r = b1.reshape(1, HIDDEN)
    b2r = b2.reshape(1, 2 * HIDDEN)
    bcr = bc.reshape(1, NUM_CLASSES)

    zeros1 = jnp.zeros((ROWS_PER_SUB,), jnp.float32)
    zeros8 = jnp.zeros((ROWS_PER_SUB, 8), jnp.float32)
    zeros16 = jnp.zeros((ROWS_PER_SUB, 16), jnp.float32)

    sc_degree, sc_edge1, sc_edge2 = _sc_kernels()
    degp = sc_degree(src, zeros1)
    g1 = _tc_prep(degp.T, posp)
    s1p = sc_edge1(src, dst, g1, zeros8)
    g2a, g2b = _tc_node1(s1p, g1, w1p, b1r)
    s2p = sc_edge2(src, dst, g2a, g2b, zeros16)
    return _tc_node2(lohi, s2p, g2a, g2b, g1, batchp, W2, b2r, Wc, bcr)


# speedup vs baseline: 1.4440x; 1.4440x over previous
import functools

import jax
import jax.numpy as jnp
from jax import lax
from jax.experimental import pallas as pl
from jax.experimental.pallas import tpu as pltpu
from jax.experimental.pallas import tpu_sc as plsc

N_NODES = 100000
N_EDGES = 3200000
NUM_GRAPHS = 64
HIDDEN = 32
NUM_CLASSES = 2

NC = 2
NS = 16
CHUNK = 128
IDX_ROWS = 16
EBLK = IDX_ROWS * CHUNK

E_PAD = ((N_EDGES + NC * NS * EBLK - 1) // (NC * NS * EBLK)) * (NC * NS * EBLK)
E_ROWS = E_PAD // CHUNK
DUMP = N_NODES
N_TAB = 100352
ROWS_PER_SUB = N_TAB // NS
R_TC = 2048
NB_TC = N_TAB // R_TC

_mesh = functools.partial(
    plsc.VectorSubcoreMesh, core_axis_name="c", subcore_axis_name="s",
    num_cores=NC, num_subcores=NS)
_SC_PARAMS = pltpu.CompilerParams(use_tc_tiling_on_sc=False)

def _pipelined_edge_block(tab_hbm, idx_s, idx_d, rows_v, acc_sh, gsem, ssem,
                          k, nrows):
    ngroup = nrows // k
    sgroups = [None] * ngroup
    gh = [pltpu.async_copy(tab_hbm.at[idx_s.at[t]], rows_v.at[t], gsem)
          for t in range(k)]
    for g in range(ngroup):
        base, boff = g * k, (g % 2) * k
        for h in gh:
            h.wait()
        sgroups[g] = [
            pltpu.async_copy(rows_v.at[boff + t],
                             acc_sh.at[idx_d.at[base + t]], ssem, add=True)
            for t in range(k)]
        if g + 1 < ngroup:
            nboff = ((g + 1) % 2) * k
            if g >= 1:
                for h in sgroups[g - 1]:
                    h.wait()
            gh = [pltpu.async_copy(tab_hbm.at[idx_s.at[base + k + t]],
                                   rows_v.at[nboff + t], gsem)
                  for t in range(k)]
    for g in range(max(0, ngroup - 2), ngroup):
        for h in sgroups[g]:
            h.wait()


@functools.lru_cache(maxsize=1)
def _sc_kernels():
    @functools.partial(
        pl.kernel,
        out_type=jax.ShapeDtypeStruct((NC, N_TAB), jnp.float32),
        mesh=_mesh(),
        compiler_params=_SC_PARAMS,
        scratch_types=[
            pltpu.VMEM((IDX_ROWS, CHUNK), jnp.int32),
            pltpu.VMEM((CHUNK,), jnp.float32),
            pltpu.VMEM_SHARED((N_TAB,), jnp.float32),
            pltpu.SemaphoreType.DMA,
        ],
    )
    def _sc_degree(src_hbm, zeros1_hbm, out_hbm, idx_v, ones_v, deg_sh, ssem):
        c = lax.axis_index("c")
        s = lax.axis_index("s")
        wid = s * NC + c

        for k in range(CHUNK // 16):
            ones_v[pl.ds(k * 16, 16)] = jnp.ones((16,), jnp.float32)

        pltpu.sync_copy(zeros1_hbm, deg_sh.at[pl.ds(s * ROWS_PER_SUB, ROWS_PER_SUB)])
        plsc.subcore_barrier()

        blocks = E_ROWS // (NC * NS)
        nblk = blocks // IDX_ROWS

        def body(b, _):
            row0 = wid * blocks + b * IDX_ROWS
            pltpu.sync_copy(src_hbm.at[pl.ds(row0, IDX_ROWS)], idx_v)
            hs = [pltpu.async_copy(ones_v, deg_sh.at[idx_v.at[j]], ssem,
                                   add=True) for j in range(IDX_ROWS)]
            for h in hs:
                h.wait()
            return 0

        lax.fori_loop(0, nblk, body, 0)
        plsc.subcore_barrier()
        pltpu.sync_copy(deg_sh.at[pl.ds(s * ROWS_PER_SUB, ROWS_PER_SUB)],
                        out_hbm.at[c, pl.ds(s * ROWS_PER_SUB, ROWS_PER_SUB)])


    @functools.partial(
        pl.kernel,
        out_type=jax.ShapeDtypeStruct((NC, N_TAB, 8), jnp.float32),
        mesh=_mesh(),
        compiler_params=_SC_PARAMS,
        scratch_types=[
            pltpu.VMEM((IDX_ROWS, CHUNK), jnp.int32),
            pltpu.VMEM((IDX_ROWS, CHUNK), jnp.int32),
            pltpu.VMEM((IDX_ROWS, CHUNK, 8), jnp.float32),
            pltpu.VMEM_SHARED((N_TAB, 8), jnp.float32),
            pltpu.SemaphoreType.DMA,
            pltpu.SemaphoreType.DMA,
        ],
    )
    def _sc_edge1(src_hbm, dst_hbm, g1_hbm, zeros8_hbm, out_hbm,
                  idx_s, idx_d, rows_v, acc_sh, gsem, ssem):
        c = lax.axis_index("c")
        s = lax.axis_index("s")
        wid = s * NC + c

        pltpu.sync_copy(zeros8_hbm, acc_sh.at[pl.ds(s * ROWS_PER_SUB, ROWS_PER_SUB)])
        plsc.subcore_barrier()

        blocks = E_ROWS // (NC * NS)
        nblk = blocks // IDX_ROWS

        def body(b, _):
            row0 = wid * blocks + b * IDX_ROWS
            pltpu.sync_copy(src_hbm.at[pl.ds(row0, IDX_ROWS)], idx_s)
            pltpu.sync_copy(dst_hbm.at[pl.ds(row0, IDX_ROWS)], idx_d)
            _pipelined_edge_block(g1_hbm, idx_s, idx_d, rows_v, acc_sh,
                                  gsem, ssem, 8, IDX_ROWS)
            return 0

        lax.fori_loop(0, nblk, body, 0)
        plsc.subcore_barrier()
        pltpu.sync_copy(acc_sh.at[pl.ds(s * ROWS_PER_SUB, ROWS_PER_SUB)],
                        out_hbm.at[c, pl.ds(s * ROWS_PER_SUB, ROWS_PER_SUB)])


    @functools.partial(
        pl.kernel,
        out_type=jax.ShapeDtypeStruct((NC, N_TAB, 16), jnp.float32),
        mesh=_mesh(),
        compiler_params=_SC_PARAMS,
        scratch_types=[
            pltpu.VMEM((IDX_ROWS, CHUNK), jnp.int32),
            pltpu.VMEM((IDX_ROWS, CHUNK), jnp.int32),
            pltpu.VMEM((8, CHUNK, 16), jnp.float32),
            pltpu.VMEM_SHARED((N_TAB, 16), jnp.float32),
            pltpu.SemaphoreType.DMA,
            pltpu.SemaphoreType.DMA,
        ],
    )
    def _sc_edge2(src_hbm, dst_hbm, g2a_hbm, g2b_hbm, zeros16_hbm, out_hbm,
                  idx_s, idx_d, rows_v, acc_sh, gsem, ssem):
        c = lax.axis_index("c")
        s = lax.axis_index("s")

        pltpu.sync_copy(zeros16_hbm, acc_sh.at[pl.ds(s * ROWS_PER_SUB, ROWS_PER_SUB)])
        plsc.subcore_barrier()

        blocks = E_ROWS // NS
        nblk = blocks // IDX_ROWS

        def run(tab_hbm):
            def body(b, _):
                row0 = s * blocks + b * IDX_ROWS
                pltpu.sync_copy(src_hbm.at[pl.ds(row0, IDX_ROWS)], idx_s)
                pltpu.sync_copy(dst_hbm.at[pl.ds(row0, IDX_ROWS)], idx_d)
                _pipelined_edge_block(tab_hbm, idx_s, idx_d, rows_v, acc_sh,
                                      gsem, ssem, 4, IDX_ROWS)
                return 0
            lax.fori_loop(0, nblk, body, 0)

        @pl.when(c == 0)
        def _():
            run(g2a_hbm)

        @pl.when(c == 1)
        def _():
            run(g2b_hbm)

        plsc.subcore_barrier()
        pltpu.sync_copy(acc_sh.at[pl.ds(s * ROWS_PER_SUB, ROWS_PER_SUB)],
                        out_hbm.at[c, pl.ds(s * ROWS_PER_SUB, ROWS_PER_SUB)])


    return _sc_degree, _sc_edge1, _sc_edge2


def _tc_prep_body(degt_ref, pos_ref, g1_ref):
    deg = degt_ref[:, 0:1] + degt_ref[:, 1:2] + 1.0
    dinv = lax.rsqrt(deg)
    g1_ref[...] = jnp.concatenate(
        [pos_ref[...] * dinv, dinv, jnp.zeros_like(pos_ref[...]), dinv], axis=1)


def _tc_prep(degt, posp):
    return pl.pallas_call(
        _tc_prep_body,
        grid=(NB_TC,),
        in_specs=[
            pl.BlockSpec((R_TC, 2), lambda i: (i, 0)),
            pl.BlockSpec((R_TC, 3), lambda i: (i, 0)),
        ],
        out_specs=pl.BlockSpec((R_TC, 8), lambda i: (i, 0)),
        out_shape=jax.ShapeDtypeStruct((N_TAB, 8), jnp.float32),
    )(degt, posp)


def _tc_node1_body(s1p_ref, g1_ref, w1_ref, b1_ref, g2a_ref, g2b_ref):
    g1 = g1_ref[...]
    dinv = g1[:, 3:4]
    aggr = dinv * (s1p_ref[0] + s1p_ref[1] + g1)
    h1 = lax.dot_general(aggr, w1_ref[...], (((1,), (1,)), ((), ())),
                         preferred_element_type=jnp.float32,
                         precision=lax.Precision.HIGHEST)
    h1 = jnp.maximum(h1 + b1_ref[...], 0.0)
    g2 = h1 * dinv
    g2a_ref[...] = g2[:, :16]
    g2b_ref[...] = g2[:, 16:]


def _tc_node1(s1p, g1, w1p, b1r):
    return pl.pallas_call(
        _tc_node1_body,
        grid=(NB_TC,),
        in_specs=[
            pl.BlockSpec((NC, R_TC, 8), lambda i: (0, i, 0)),
            pl.BlockSpec((R_TC, 8), lambda i: (i, 0)),
            pl.BlockSpec((HIDDEN, 8), lambda i: (0, 0)),
            pl.BlockSpec((1, HIDDEN), lambda i: (0, 0)),
        ],
        out_specs=[
            pl.BlockSpec((R_TC, 16), lambda i: (i, 0)),
            pl.BlockSpec((R_TC, 16), lambda i: (i, 0)),
        ],
        out_shape=[
            jax.ShapeDtypeStruct((N_TAB, 16), jnp.float32),
            jax.ShapeDtypeStruct((N_TAB, 16), jnp.float32),
        ],
    )(s1p, g1, w1p, b1r)


def _tc_node2_body(lohi_ref, s2p_ref, g2a_ref, g2b_ref, g1_ref, batch_ref,
                   w2_ref, b2_ref, wc_ref, bc_ref, out_ref, pool_ref):
    i = pl.program_id(0)

    @pl.when(i == 0)
    def _():
        pool_ref[...] = jnp.full((NUM_GRAPHS, 2 * HIDDEN), -jnp.inf,
                                 jnp.float32)

    g2 = jnp.concatenate([g2a_ref[...], g2b_ref[...]], axis=1)
    s2 = jnp.concatenate([s2p_ref[0], s2p_ref[1]], axis=1)
    dinv = g1_ref[:, 3:4]
    aggr = dinv * (s2 + g2)
    h2 = lax.dot_general(aggr, w2_ref[...], (((1,), (1,)), ((), ())),
                         preferred_element_type=jnp.float32,
                         precision=lax.Precision.HIGHEST)
    h2 = jnp.maximum(h2 + b2_ref[...], 0.0)

    bb = batch_ref[...]
    lo = lohi_ref[i, 0]
    hi = lohi_ref[i, 1]
    for g in range(NUM_GRAPHS):
        @pl.when((lo <= g) & (g <= hi))
        def _():
            sel = bb == g
            m = jnp.max(jnp.where(sel, h2, -jnp.inf), axis=0, keepdims=True)
            pool_ref[g:g + 1, :] = jnp.maximum(pool_ref[g:g + 1, :], m)

    @pl.when(i == NB_TC - 1)
    def _():
        out = lax.dot_general(pool_ref[...], wc_ref[...],
                              (((1,), (1,)), ((), ())),
                              preferred_element_type=jnp.float32,
                              precision=lax.Precision.HIGHEST)
        out_ref[...] = out + bc_ref[...]


def _tc_node2(lohi, s2p, g2a, g2b, g1, batchp, w2, b2r, wc, bcr):
    return pl.pallas_call(
        _tc_node2_body,
        grid=(NB_TC,),
        in_specs=[
            pl.BlockSpec(memory_space=pltpu.SMEM),
            pl.BlockSpec((NC, R_TC, 16), lambda i: (0, i, 0)),
            pl.BlockSpec((R_TC, 16), lambda i: (i, 0)),
            pl.BlockSpec((R_TC, 16), lambda i: (i, 0)),
            pl.BlockSpec((R_TC, 8), lambda i: (i, 0)),
            pl.BlockSpec((R_TC, 1), lambda i: (i, 0)),
            pl.BlockSpec((2 * HIDDEN, HIDDEN), lambda i: (0, 0)),
            pl.BlockSpec((1, 2 * HIDDEN), lambda i: (0, 0)),
            pl.BlockSpec((NUM_CLASSES, 2 * HIDDEN), lambda i: (0, 0)),
            pl.BlockSpec((1, NUM_CLASSES), lambda i: (0, 0)),
        ],
        out_specs=pl.BlockSpec((NUM_GRAPHS, NUM_CLASSES), lambda i: (0, 0)),
        out_shape=jax.ShapeDtypeStruct((NUM_GRAPHS, NUM_CLASSES), jnp.float32),
        scratch_shapes=[pltpu.VMEM((NUM_GRAPHS, 2 * HIDDEN), jnp.float32)],
    )(lohi, s2p, g2a, g2b, g1, batchp, w2, b2r, wc, bcr)


def kernel(pos, edge_index, batch, W1, b1, W2, b2, Wc, bc):
    pad_e = E_PAD - N_EDGES
    src = jnp.concatenate(
        [edge_index[0].astype(jnp.int32),
         jnp.full((pad_e,), DUMP, jnp.int32)]).reshape(E_ROWS, CHUNK)
    dst = jnp.concatenate(
        [edge_index[1].astype(jnp.int32),
         jnp.full((pad_e,), DUMP, jnp.int32)]).reshape(E_ROWS, CHUNK)
    posp = jnp.pad(pos, ((0, N_TAB - N_NODES), (0, 0)))
    batchp = jnp.pad(batch.astype(jnp.int32), (0, N_TAB - N_NODES),
                     constant_values=NUM_GRAPHS).reshape(N_TAB, 1)
    lohi = jnp.stack([batchp[::R_TC, 0], batchp[R_TC - 1::R_TC, 0]], axis=1)
    w1p = jnp.pad(W1, ((0, 0), (0, 5)))
    b1r = b1.reshape(1, HIDDEN)
    b2r = b2.reshape(1, 2 * HIDDEN)
    bcr = bc.reshape(1, NUM_CLASSES)

    zeros1 = jnp.zeros((ROWS_PER_SUB,), jnp.float32)
    zeros8 = jnp.zeros((ROWS_PER_SUB, 8), jnp.float32)
    zeros16 = jnp.zeros((ROWS_PER_SUB, 16), jnp.float32)

    sc_degree, sc_edge1, sc_edge2 = _sc_kernels()
    degp = sc_degree(src, zeros1)
    g1 = _tc_prep(degp.T, posp)
    s1p = sc_edge1(src, dst, g1, zeros8)
    g2a, g2b = _tc_node1(s1p, g1, w1p, b1r)
    s2p = sc_edge2(src, dst, g2a, g2b, zeros16)
    return _tc_node2(lohi, s2p, g2a, g2b, g1, batchp, W2, b2r, Wc, bcr)
